# baseline (speedup 1.0000x reference)
import jax
import jax.numpy as jnp
from jax.experimental import pallas as pl
from jax.experimental.pallas import tpu as pltpu

_B = 16
_MAX_SEQ = 2048
_H = 8
_D = 128
_PAGE = 256
_T = 32
_PAGES_PER_SEQ = _MAX_SEQ // _PAGE
_NUM_PAGES = _B * _PAGES_PER_SEQ
_ROWS = _NUM_PAGES * _PAGE


def _body(tp_ref, s0_ref, x_hbm, xc_hbm, xo_hbm, xtok, tok_sem):
    del xc_hbm
    cp = pltpu.make_async_copy(x_hbm, xtok, tok_sem)
    cp.start()
    cp.wait()
    toks = []
    for b in range(_B):
        dst = pl.multiple_of(tp_ref[b] * _PAGE + s0_ref[b], 8)
        toks.append(pltpu.make_async_copy(
            xtok.at[pl.ds(b * _T, _T)], xo_hbm.at[pl.ds(dst, _T)], tok_sem))
    for c in toks:
        c.start()
    for c in toks:
        c.wait()


def _scatter_into_copy(tp, s0, x2, xc2):
    return pl.pallas_call(
        _body,
        grid=(),
        in_specs=[
            pl.BlockSpec(memory_space=pltpu.SMEM),
            pl.BlockSpec(memory_space=pltpu.SMEM),
            pl.BlockSpec(memory_space=pl.ANY),
            pl.BlockSpec(memory_space=pl.ANY),
        ],
        out_specs=pl.BlockSpec(memory_space=pl.ANY),
        out_shape=jax.ShapeDtypeStruct((_ROWS, _H * _D), xc2.dtype),
        input_output_aliases={3: 0},
        scratch_shapes=[
            pltpu.VMEM((_B * _T, _H * _D), x2.dtype),
            pltpu.SemaphoreType.DMA,
        ],
    )(tp, s0, x2, xc2)


def kernel(k, v, k_cache, v_cache, page_table, cache_seqlens):
    k2 = k.reshape(_B * _T, _H * _D)
    v2 = v.reshape(_B * _T, _H * _D)
    kc2 = k_cache.reshape(_ROWS, _H * _D)
    vc2 = v_cache.reshape(_ROWS, _H * _D)

    pos0 = cache_seqlens
    pg = pos0 // _PAGE
    tp = jnp.take_along_axis(page_table, pg[:, None], axis=1)[:, 0]
    s0 = pos0 % _PAGE

    ko2 = _scatter_into_copy(tp, s0, k2, kc2)
    vo2 = _scatter_into_copy(tp, s0, v2, vc2)

    k_cache_new = ko2.reshape(_NUM_PAGES, _PAGE, _H, _D)
    v_cache_new = vo2.reshape(_NUM_PAGES, _PAGE, _H, _D)
    return (k_cache_new, v_cache_new, cache_seqlens + _T)
